# baseline (speedup 1.0000x reference)
import functools

import jax
import jax.numpy as jnp
from jax import lax
from jax.experimental import pallas as pl
from jax.experimental.pallas import tpu as pltpu
from jax.experimental.pallas import tpu_sc as plsc

N = 100000
E = 1600000
EP = 1638400
C = 11392
NCH = 9
CH0 = 5
NPAD = NCH * C
SHIFT = 14
DMASK = (1 << SHIFT) - 1
SP = (C + 128) // 16
WS = C // 16
ET = EP // 16
B = 2048
NBLK = ET // B
NV = B // 16

_MESH = plsc.VectorSubcoreMesh(core_axis_name="c", subcore_axis_name="s")
_DN = lax.GatherDimensionNumbers(
    offset_dims=(), collapsed_slice_dims=(0,), start_index_map=(0,))


def _lane_gather(x, idx):
    return lax.gather(x, idx[:, None], _DN, slice_sizes=(1,),
                      mode=lax.GatherScatterMode.PROMISE_IN_BOUNDS)


def _prefix_incl(x, io16):
    for k in (1, 2, 4, 8):
        sh = _lane_gather(x, jnp.maximum(io16 - k, 0))
        x = x + jnp.where(io16 >= k, sh, 0)
    return x


def _compact(vals, mi, io16):
    pref = _prefix_incl(mi, io16)
    total = pref[15]
    pos = jnp.zeros((16,), jnp.int32)
    r = io16 + 1
    for b in (8, 4, 2, 1):
        npos = pos + b
        pv = _lane_gather(pref, npos - 1)
        pos = jnp.where(pv < r, npos, pos)
    return _lane_gather(vals, jnp.minimum(pos, 15)), total


def _zero_stripe(zeros_hbm, acc, base, rows):
    for j in range(rows // 128):
        pltpu.sync_copy(zeros_hbm, acc.at[pl.ds(base + j * 128, 128)])
    rem = rows % 128
    if rem:
        pltpu.sync_copy(zeros_hbm.at[pl.ds(0, rem)],
                        acc.at[pl.ds(base + (rows // 128) * 128, rem)])


def _writeback(acc, out_hbm, s, lo):
    wbase = s * WS
    for j in range(WS // 128):
        pltpu.sync_copy(acc.at[pl.ds(wbase + j * 128, 128)],
                        out_hbm.at[pl.ds(lo + wbase + j * 128, 128)])
    wrem = WS % 128
    if wrem:
        woff = wbase + (WS // 128) * 128
        pltpu.sync_copy(acc.at[pl.ds(woff, wrem)],
                        out_hbm.at[pl.ds(lo + woff, wrem)])


@functools.partial(
    pl.kernel,
    mesh=_MESH,
    out_type=jax.ShapeDtypeStruct((NPAD, 128), jnp.float32),
    scratch_types=(
        [pltpu.VMEM((B,), jnp.int32)] * 2
        + [pltpu.VMEM((64,), jnp.int32)]
        + [pltpu.VMEM((64,), jnp.int32)] * 8
        + [pltpu.VMEM((64, 128), jnp.float32)] * 4
        + [pltpu.VMEM_SHARED((C + 128, 128), jnp.float32)]
        + [pltpu.SemaphoreType.DMA] * 8
    ),
)
def _prop(z_hbm, src_hbm, dst_hbm, zeros_hbm, out_hbm,
          srcblk, dstblk, cbuf,
          csrc0, cdst0, csrc1, cdst1, csrc2, cdst2, csrc3, cdst3,
          rb0, rb1, rb2, rb3, acc,
          sg0, sg1, sg2, sg3, ss0, ss1, ss2, ss3):
    c = lax.axis_index("c")
    s = lax.axis_index("s")
    nmy = jnp.where(c == 0, CH0, NCH - CH0)
    io16 = lax.iota(jnp.int32, 16)
    slots = ((csrc0, cdst0, rb0, sg0, ss0), (csrc1, cdst1, rb1, sg1, ss1),
             (csrc2, cdst2, rb2, sg2, ss2), (csrc3, cdst3, rb3, sg3, ss3))

    def each_slot(sel, fn):
        for t in range(4):
            @pl.when(sel == t)
            def _():
                fn(*slots[t])

    def start_scatter(csrc, cdst, rb, sg, ss):
        pltpu.make_async_copy(z_hbm.at[csrc], rb, sg).wait()
        pltpu.async_copy(rb, acc.at[cdst], ss, add=True)

    def wait_scatter(csrc, cdst, rb, sg, ss):
        pltpu.make_async_copy(rb, acc.at[cdst], ss).wait()

    def load_gather(n):
        def fn(csrc, cdst, rb, sg, ss):
            for j in range(4):
                v = cbuf[pl.ds(j * 16, 16)]
                valid = (io16 + (j * 16)) < n
                csrc[pl.ds(j * 16, 16)] = jnp.where(
                    valid, (v >> SHIFT) & 0x1FFFF, 0)
                cdst[pl.ds(j * 16, 16)] = jnp.where(valid, v & DMASK, C)
            pltpu.make_async_copy(z_hbm.at[csrc], rb, sg).start()
        return fn

    def flush(n, fc):
        @pl.when(fc >= 1)
        def _():
            each_slot((fc - 1) & 3, start_scatter)

        @pl.when(fc >= 4)
        def _():
            each_slot(fc & 3, wait_scatter)

        each_slot(fc & 3, load_gather(n))

    def chunk_body(kk, carry):
        @pl.when(kk < nmy)
        def _():
            chunk = 2 * kk + c
            lo = chunk * C
            _zero_stripe(zeros_hbm, acc, s * SP, SP)
            plsc.subcore_barrier()

            def blk_body(b, cur_fc):
                off = s * ET + b * B
                pltpu.sync_copy(src_hbm.at[pl.ds(off, B)], srcblk)
                pltpu.sync_copy(dst_hbm.at[pl.ds(off, B)], dstblk)

                def v_body(v, cur_fc2):
                    cur2, fc = cur_fc2
                    sv = srcblk[pl.ds(v * 16, 16)]
                    dv = dstblk[pl.ds(v * 16, 16)]
                    dl = dv - lo
                    m = (dl >= 0) & (dl < C)
                    packed = (sv << SHIFT) | (dl & DMASK)
                    comp, total = _compact(packed, jnp.where(m, 1, 0), io16)
                    cbuf[pl.ds(cur2, 16)] = comp
                    cur3 = cur2 + total
                    fl = cur3 > 48

                    @pl.when(fl)
                    def _():
                        flush(cur3, fc)

                    return (jnp.where(fl, 0, cur3),
                            jnp.where(fl, fc + 1, fc))

                return lax.fori_loop(0, NV, v_body, cur_fc)

            cur, fc = lax.fori_loop(0, NBLK, blk_body, (0, 0))

            @pl.when(cur > 0)
            def _():
                flush(cur, fc)

            nfl = jnp.where(cur > 0, fc + 1, fc)

            @pl.when(nfl >= 1)
            def _():
                each_slot((nfl - 1) & 3, start_scatter)
            for t in range(4):
                @pl.when(nfl >= t + 1)
                def _():
                    wait_scatter(*slots[t])

            plsc.subcore_barrier()
            _writeback(acc, out_hbm, s, lo)
            plsc.subcore_barrier()
        return carry

    lax.fori_loop(0, CH0, chunk_body, 0)


@functools.partial(
    pl.kernel,
    mesh=_MESH,
    out_type=jax.ShapeDtypeStruct((NPAD, 128), jnp.float32),
    scratch_types=[
        pltpu.VMEM((B,), jnp.int32),
        pltpu.VMEM((128,), jnp.int32),
        pltpu.VMEM((128,), jnp.int32),
        pltpu.VMEM((128, 128), jnp.float32),
        pltpu.VMEM_SHARED((C + 128, 128), jnp.float32),
    ],
)
def _degree(ones_hbm, dst_hbm, zeros_hbm, out_hbm,
            dstblk, cbuf, cdst, onesbuf, acc):
    c = lax.axis_index("c")
    s = lax.axis_index("s")
    nmy = jnp.where(c == 0, CH0, NCH - CH0)
    io16 = lax.iota(jnp.int32, 16)

    pltpu.sync_copy(ones_hbm, onesbuf)

    def flush(n):
        for j in range(8):
            v = cbuf[pl.ds(j * 16, 16)]
            valid = (io16 + (j * 16)) < n
            cdst[pl.ds(j * 16, 16)] = jnp.where(valid, v, C)
        pltpu.sync_copy(onesbuf, acc.at[cdst], add=True)

    def chunk_body(kk, carry):
        @pl.when(kk < nmy)
        def _():
            chunk = 2 * kk + c
            lo = chunk * C
            _zero_stripe(zeros_hbm, acc, s * SP, SP)
            plsc.subcore_barrier()

            def blk_body(b, cur):
                off = s * ET + b * B
                pltpu.sync_copy(dst_hbm.at[pl.ds(off, B)], dstblk)

                def v_body(v, cur2):
                    dv = dstblk[pl.ds(v * 16, 16)]
                    dl = dv - lo
                    m = (dl >= 0) & (dl < C)
                    comp, total = _compact(dl, jnp.where(m, 1, 0), io16)
                    cbuf[pl.ds(cur2, 16)] = comp
                    cur3 = cur2 + total
                    fl = cur3 > 112

                    @pl.when(fl)
                    def _():
                        flush(cur3)

                    return jnp.where(fl, 0, cur3)

                return lax.fori_loop(0, NV, v_body, cur)

            cur = lax.fori_loop(0, NBLK, blk_body, 0)

            @pl.when(cur > 0)
            def _():
                flush(cur)

            plsc.subcore_barrier()
            _writeback(acc, out_hbm, s, lo)
            plsc.subcore_barrier()
        return carry

    lax.fori_loop(0, CH0, chunk_body, 0)


_R = 2048
_G = (N + _R - 1) // _R


def _tc_encode(x, dp, wih_t, whh_t, b_ih, b_hh, sW, sb):
    def body(x_ref, dp_ref, wih_ref, whh_ref, bih_ref, bhh_ref,
             sW_ref, sb_ref, z1_ref, dinv_ref):
        xb = x_ref[...]
        deg = dp_ref[...][:, 0] + 1.0
        dinv = lax.rsqrt(deg)
        pre = xb[:, :16] @ sW_ref[...] + sb_ref[...]
        s_enc = 0.5 * pre * (1.0 + lax.erf(pre * 0.7071067811865476))
        h = jnp.zeros((_R, 64), jnp.float32)
        for t in range(6):
            xt = xb[:, 16 + t:17 + t]
            gi = xt * wih_ref[...] + bih_ref[...]
            gh = h @ whh_ref[...] + bhh_ref[...]
            r = jax.nn.sigmoid(gi[:, :64] + gh[:, :64])
            z = jax.nn.sigmoid(gi[:, 64:128] + gh[:, 64:128])
            nn_ = jnp.tanh(gi[:, 128:] + r * gh[:, 128:])
            h = (1.0 - z) * nn_ + z * h
        hcat = jnp.concatenate(
            [h, s_enc, jnp.zeros((_R, 32), jnp.float32)], axis=1)
        z1_ref[...] = dinv[:, None] * hcat
        dinv_ref[...] = dinv[:, None]

    return pl.pallas_call(
        body,
        grid=(_G,),
        out_shape=(jax.ShapeDtypeStruct((N, 128), jnp.float32),
                   jax.ShapeDtypeStruct((N, 1), jnp.float32)),
        in_specs=[
            pl.BlockSpec((_R, 34), lambda i: (i, 0)),
            pl.BlockSpec((_R, 128), lambda i: (i, 0)),
            pl.BlockSpec((1, 192), lambda i: (0, 0)),
            pl.BlockSpec((64, 192), lambda i: (0, 0)),
            pl.BlockSpec((1, 192), lambda i: (0, 0)),
            pl.BlockSpec((1, 192), lambda i: (0, 0)),
            pl.BlockSpec((16, 32), lambda i: (0, 0)),
            pl.BlockSpec((1, 32), lambda i: (0, 0)),
        ],
        out_specs=(pl.BlockSpec((_R, 128), lambda i: (i, 0)),
                   pl.BlockSpec((_R, 1), lambda i: (i, 0))),
    )(x, dp, wih_t, whh_t, b_ih, b_hh, sW, sb)


def _tc_mid(acc1, z1, dinv, W1, b1, W2):
    def body(acc_ref, z1_ref, dinv_ref, W1_ref, b1_ref, W2_ref, z2_ref):
        p1 = dinv_ref[...] * (acc_ref[...][:, :96] + z1_ref[...][:, :96])
        h1 = jnp.maximum(p1 @ W1_ref[...] + b1_ref[...], 0.0)
        z2 = dinv_ref[...] * (h1 @ W2_ref[...])
        z2_ref[...] = jnp.concatenate(
            [z2, jnp.zeros((_R, 64), jnp.float32)], axis=1)

    return pl.pallas_call(
        body,
        grid=(_G,),
        out_shape=jax.ShapeDtypeStruct((N, 128), jnp.float32),
        in_specs=[
            pl.BlockSpec((_R, 128), lambda i: (i, 0)),
            pl.BlockSpec((_R, 128), lambda i: (i, 0)),
            pl.BlockSpec((_R, 1), lambda i: (i, 0)),
            pl.BlockSpec((96, 128), lambda i: (0, 0)),
            pl.BlockSpec((1, 128), lambda i: (0, 0)),
            pl.BlockSpec((128, 64), lambda i: (0, 0)),
        ],
        out_specs=pl.BlockSpec((_R, 128), lambda i: (i, 0)),
    )(acc1, z1, dinv, W1, b1, W2)


def _tc_head(acc2, z2, dinv, b2, head_W, head_b):
    def body(acc_ref, z2_ref, dinv_ref, b2_ref, hw_ref, hb_ref, out_ref):
        h2 = jnp.maximum(
            dinv_ref[...] * (acc_ref[...][:, :64] + z2_ref[...][:, :64])
            + b2_ref[...], 0.0)
        out_ref[...] = jax.nn.sigmoid(h2 @ hw_ref[...] + hb_ref[0, 0])

    return pl.pallas_call(
        body,
        grid=(_G,),
        out_shape=jax.ShapeDtypeStruct((N, 1), jnp.float32),
        in_specs=[
            pl.BlockSpec((_R, 128), lambda i: (i, 0)),
            pl.BlockSpec((_R, 128), lambda i: (i, 0)),
            pl.BlockSpec((_R, 1), lambda i: (i, 0)),
            pl.BlockSpec((1, 64), lambda i: (0, 0)),
            pl.BlockSpec((64, 1), lambda i: (0, 0)),
            pl.BlockSpec((1, 1), lambda i: (0, 0)),
        ],
        out_specs=pl.BlockSpec((_R, 1), lambda i: (i, 0)),
    )(acc2, z2, dinv, b2, head_W, head_b)


def kernel(x, edge_index, gru_W_ih, gru_W_hh, gru_b_ih, gru_b_hh, static_W,
           static_b, gcn1_W, gcn1_b, gcn2_W, gcn2_b, head_W, head_b):
    pad = EP - E
    src = jnp.concatenate([edge_index[0], jnp.zeros((pad,), jnp.int32)])
    dst = jnp.concatenate([edge_index[1], jnp.full((pad,), NPAD, jnp.int32)])

    zeros128 = jnp.zeros((128, 128), jnp.float32)
    ones128 = zeros128.at[:, 0].set(1.0)
    dp = _degree(ones128, dst, zeros128)
    z1, dinv = _tc_encode(
        x, dp,
        gru_W_ih.reshape(1, 192), gru_W_hh.T,
        gru_b_ih.reshape(1, 192), gru_b_hh.reshape(1, 192),
        static_W, static_b.reshape(1, 32))

    acc1 = _prop(z1, src, dst, zeros128)
    z2 = _tc_mid(acc1, z1, dinv, gcn1_W, gcn1_b.reshape(1, 128), gcn2_W)

    acc2 = _prop(z2, src, dst, zeros128)
    out = _tc_head(acc2, z2, dinv, gcn2_b.reshape(1, 64), head_W,
                   head_b.reshape(1, 1))
    return out[:, 0]


# speedup vs baseline: 2.0471x; 2.0471x over previous
import functools

import jax
import jax.numpy as jnp
from jax import lax
from jax.experimental import pallas as pl
from jax.experimental.pallas import tpu as pltpu
from jax.experimental.pallas import tpu_sc as plsc

N = 100000
E = 1600000
EP = 1638400
C = 13568
NCH = 8
CH0 = 4
NPAD = NCH * C
SHIFT = 14
DMASK = (1 << SHIFT) - 1
SP = (C + 128) // 16
WS = C // 16
ET = EP // 16
B = 2048
NBLK = ET // B
NV = B // 16

_MESH = plsc.VectorSubcoreMesh(core_axis_name="c", subcore_axis_name="s")
_DN = lax.GatherDimensionNumbers(
    offset_dims=(), collapsed_slice_dims=(0,), start_index_map=(0,))


def _lane_gather(x, idx):
    return lax.gather(x, idx[:, None], _DN, slice_sizes=(1,),
                      mode=lax.GatherScatterMode.PROMISE_IN_BOUNDS)


def _prefix_incl(x, io16):
    for k in (1, 2, 4, 8):
        sh = _lane_gather(x, jnp.maximum(io16 - k, 0))
        x = x + jnp.where(io16 >= k, sh, 0)
    return x


def _compact(vals, mi, io16):
    pref = _prefix_incl(mi, io16)
    total = pref[15]
    pos = jnp.zeros((16,), jnp.int32)
    r = io16 + 1
    for b in (8, 4, 2, 1):
        npos = pos + b
        pv = _lane_gather(pref, npos - 1)
        pos = jnp.where(pv < r, npos, pos)
    return _lane_gather(vals, jnp.minimum(pos, 15)), total


def _zero_stripe(zeros_hbm, acc, base, rows):
    for j in range(rows // 128):
        pltpu.sync_copy(zeros_hbm, acc.at[pl.ds(base + j * 128, 128)])
    rem = rows % 128
    if rem:
        pltpu.sync_copy(zeros_hbm.at[pl.ds(0, rem)],
                        acc.at[pl.ds(base + (rows // 128) * 128, rem)])


def _writeback(acc, out_hbm, s, lo):
    wbase = s * WS
    for j in range(WS // 128):
        pltpu.sync_copy(acc.at[pl.ds(wbase + j * 128, 128)],
                        out_hbm.at[pl.ds(lo + wbase + j * 128, 128)])
    wrem = WS % 128
    if wrem:
        woff = wbase + (WS // 128) * 128
        pltpu.sync_copy(acc.at[pl.ds(woff, wrem)],
                        out_hbm.at[pl.ds(lo + woff, wrem)])


@functools.partial(
    pl.kernel,
    mesh=_MESH,
    out_type=jax.ShapeDtypeStruct((NPAD, 128), jnp.float32),
    scratch_types=[
        pltpu.VMEM((B,), jnp.int32),
        pltpu.VMEM((B,), jnp.int32),
        pltpu.VMEM((128,), jnp.int32),
        pltpu.VMEM((128,), jnp.int32),
        pltpu.VMEM((128,), jnp.int32),
        pltpu.VMEM((128, 128), jnp.float32),
        pltpu.VMEM_SHARED((C + 128, 128), jnp.float32),
        pltpu.SemaphoreType.DMA,
    ],
)
def _prop(z_hbm, src_hbm, dst_hbm, zeros_hbm, out_hbm,
          srcblk, dstblk, cbuf, csrc, cdst, rowbuf, acc, sem):
    c = lax.axis_index("c")
    s = lax.axis_index("s")
    nmy = jnp.where(c == 0, CH0, NCH - CH0)
    io16 = lax.iota(jnp.int32, 16)

    def flush(n):
        for j in range(8):
            v = cbuf[pl.ds(j * 16, 16)]
            valid = (io16 + (j * 16)) < n
            csrc[pl.ds(j * 16, 16)] = jnp.where(
                valid, (v >> SHIFT) & 0x1FFFF, 0)
            cdst[pl.ds(j * 16, 16)] = jnp.where(valid, v & DMASK, C)
        pltpu.async_copy(z_hbm.at[csrc], rowbuf, sem).wait()
        pltpu.sync_copy(rowbuf, acc.at[cdst], add=True)

    def chunk_body(kk, carry):
        @pl.when(kk < nmy)
        def _():
            chunk = 2 * kk + c
            lo = chunk * C
            _zero_stripe(zeros_hbm, acc, s * SP, SP)
            plsc.subcore_barrier()

            def blk_body(b, cur):
                off = s * ET + b * B
                pltpu.sync_copy(src_hbm.at[pl.ds(off, B)], srcblk)
                pltpu.sync_copy(dst_hbm.at[pl.ds(off, B)], dstblk)

                def v_body(v, cur2):
                    sv = srcblk[pl.ds(v * 16, 16)]
                    dv = dstblk[pl.ds(v * 16, 16)]
                    dl = dv - lo
                    m = (dl >= 0) & (dl < C)
                    packed = (sv << SHIFT) | (dl & DMASK)
                    comp, total = _compact(packed, jnp.where(m, 1, 0), io16)
                    cbuf[pl.ds(cur2, 16)] = comp
                    cur3 = cur2 + total
                    fl = cur3 > 112

                    @pl.when(fl)
                    def _():
                        flush(cur3)

                    return jnp.where(fl, 0, cur3)

                return lax.fori_loop(0, NV, v_body, cur)

            cur = lax.fori_loop(0, NBLK, blk_body, 0)

            @pl.when(cur > 0)
            def _():
                flush(cur)

            plsc.subcore_barrier()
            _writeback(acc, out_hbm, s, lo)
            plsc.subcore_barrier()
        return carry

    lax.fori_loop(0, CH0, chunk_body, 0)


@functools.partial(
    pl.kernel,
    mesh=_MESH,
    out_type=jax.ShapeDtypeStruct((NPAD, 128), jnp.float32),
    scratch_types=[
        pltpu.VMEM((B,), jnp.int32),
        pltpu.VMEM((128,), jnp.int32),
        pltpu.VMEM((128,), jnp.int32),
        pltpu.VMEM((128, 128), jnp.float32),
        pltpu.VMEM_SHARED((C + 128, 128), jnp.float32),
    ],
)
def _degree(ones_hbm, dst_hbm, zeros_hbm, out_hbm,
            dstblk, cbuf, cdst, onesbuf, acc):
    c = lax.axis_index("c")
    s = lax.axis_index("s")
    nmy = jnp.where(c == 0, CH0, NCH - CH0)
    io16 = lax.iota(jnp.int32, 16)

    pltpu.sync_copy(ones_hbm, onesbuf)

    def flush(n):
        for j in range(8):
            v = cbuf[pl.ds(j * 16, 16)]
            valid = (io16 + (j * 16)) < n
            cdst[pl.ds(j * 16, 16)] = jnp.where(valid, v, C)
        pltpu.sync_copy(onesbuf, acc.at[cdst], add=True)

    def chunk_body(kk, carry):
        @pl.when(kk < nmy)
        def _():
            chunk = 2 * kk + c
            lo = chunk * C
            _zero_stripe(zeros_hbm, acc, s * SP, SP)
            plsc.subcore_barrier()

            def blk_body(b, cur):
                off = s * ET + b * B
                pltpu.sync_copy(dst_hbm.at[pl.ds(off, B)], dstblk)

                def v_body(v, cur2):
                    dv = dstblk[pl.ds(v * 16, 16)]
                    dl = dv - lo
                    m = (dl >= 0) & (dl < C)
                    comp, total = _compact(dl, jnp.where(m, 1, 0), io16)
                    cbuf[pl.ds(cur2, 16)] = comp
                    cur3 = cur2 + total
                    fl = cur3 > 112

                    @pl.when(fl)
                    def _():
                        flush(cur3)

                    return jnp.where(fl, 0, cur3)

                return lax.fori_loop(0, NV, v_body, cur)

            cur = lax.fori_loop(0, NBLK, blk_body, 0)

            @pl.when(cur > 0)
            def _():
                flush(cur)

            plsc.subcore_barrier()
            _writeback(acc, out_hbm, s, lo)
            plsc.subcore_barrier()
        return carry

    lax.fori_loop(0, CH0, chunk_body, 0)


_R = 2048
_G = (N + _R - 1) // _R


def _tc_encode(x, dp, wih_t, whh_t, b_ih, b_hh, sW, sb):
    def body(x_ref, dp_ref, wih_ref, whh_ref, bih_ref, bhh_ref,
             sW_ref, sb_ref, z1_ref, dinv_ref):
        xb = x_ref[...]
        deg = dp_ref[...][:, 0] + 1.0
        dinv = lax.rsqrt(deg)
        pre = xb[:, :16] @ sW_ref[...] + sb_ref[...]
        s_enc = 0.5 * pre * (1.0 + lax.erf(pre * 0.7071067811865476))
        h = jnp.zeros((_R, 64), jnp.float32)
        for t in range(6):
            xt = xb[:, 16 + t:17 + t]
            gi = xt * wih_ref[...] + bih_ref[...]
            gh = h @ whh_ref[...] + bhh_ref[...]
            r = jax.nn.sigmoid(gi[:, :64] + gh[:, :64])
            z = jax.nn.sigmoid(gi[:, 64:128] + gh[:, 64:128])
            nn_ = jnp.tanh(gi[:, 128:] + r * gh[:, 128:])
            h = (1.0 - z) * nn_ + z * h
        hcat = jnp.concatenate(
            [h, s_enc, jnp.zeros((_R, 32), jnp.float32)], axis=1)
        z1_ref[...] = dinv[:, None] * hcat
        dinv_ref[...] = dinv[:, None]

    return pl.pallas_call(
        body,
        grid=(_G,),
        out_shape=(jax.ShapeDtypeStruct((N, 128), jnp.float32),
                   jax.ShapeDtypeStruct((N, 1), jnp.float32)),
        in_specs=[
            pl.BlockSpec((_R, 34), lambda i: (i, 0)),
            pl.BlockSpec((_R, 128), lambda i: (i, 0)),
            pl.BlockSpec((1, 192), lambda i: (0, 0)),
            pl.BlockSpec((64, 192), lambda i: (0, 0)),
            pl.BlockSpec((1, 192), lambda i: (0, 0)),
            pl.BlockSpec((1, 192), lambda i: (0, 0)),
            pl.BlockSpec((16, 32), lambda i: (0, 0)),
            pl.BlockSpec((1, 32), lambda i: (0, 0)),
        ],
        out_specs=(pl.BlockSpec((_R, 128), lambda i: (i, 0)),
                   pl.BlockSpec((_R, 1), lambda i: (i, 0))),
    )(x, dp, wih_t, whh_t, b_ih, b_hh, sW, sb)


def _tc_mid(acc1, z1, dinv, W1, b1, W2):
    def body(acc_ref, z1_ref, dinv_ref, W1_ref, b1_ref, W2_ref, z2_ref):
        p1 = dinv_ref[...] * (acc_ref[...][:, :96] + z1_ref[...][:, :96])
        h1 = jnp.maximum(p1 @ W1_ref[...] + b1_ref[...], 0.0)
        z2 = dinv_ref[...] * (h1 @ W2_ref[...])
        z2_ref[...] = jnp.concatenate(
            [z2, jnp.zeros((_R, 64), jnp.float32)], axis=1)

    return pl.pallas_call(
        body,
        grid=(_G,),
        out_shape=jax.ShapeDtypeStruct((N, 128), jnp.float32),
        in_specs=[
            pl.BlockSpec((_R, 128), lambda i: (i, 0)),
            pl.BlockSpec((_R, 128), lambda i: (i, 0)),
            pl.BlockSpec((_R, 1), lambda i: (i, 0)),
            pl.BlockSpec((96, 128), lambda i: (0, 0)),
            pl.BlockSpec((1, 128), lambda i: (0, 0)),
            pl.BlockSpec((128, 64), lambda i: (0, 0)),
        ],
        out_specs=pl.BlockSpec((_R, 128), lambda i: (i, 0)),
    )(acc1, z1, dinv, W1, b1, W2)


def _tc_head(acc2, z2, dinv, b2, head_W, head_b):
    def body(acc_ref, z2_ref, dinv_ref, b2_ref, hw_ref, hb_ref, out_ref):
        h2 = jnp.maximum(
            dinv_ref[...] * (acc_ref[...][:, :64] + z2_ref[...][:, :64])
            + b2_ref[...], 0.0)
        out_ref[...] = jax.nn.sigmoid(h2 @ hw_ref[...] + hb_ref[0, 0])

    return pl.pallas_call(
        body,
        grid=(_G,),
        out_shape=jax.ShapeDtypeStruct((N, 1), jnp.float32),
        in_specs=[
            pl.BlockSpec((_R, 128), lambda i: (i, 0)),
            pl.BlockSpec((_R, 128), lambda i: (i, 0)),
            pl.BlockSpec((_R, 1), lambda i: (i, 0)),
            pl.BlockSpec((1, 64), lambda i: (0, 0)),
            pl.BlockSpec((64, 1), lambda i: (0, 0)),
            pl.BlockSpec((1, 1), lambda i: (0, 0)),
        ],
        out_specs=pl.BlockSpec((_R, 1), lambda i: (i, 0)),
    )(acc2, z2, dinv, b2, head_W, head_b)


def kernel(x, edge_index, gru_W_ih, gru_W_hh, gru_b_ih, gru_b_hh, static_W,
           static_b, gcn1_W, gcn1_b, gcn2_W, gcn2_b, head_W, head_b):
    pad = EP - E
    src = jnp.concatenate([edge_index[0], jnp.zeros((pad,), jnp.int32)])
    dst = jnp.concatenate([edge_index[1], jnp.full((pad,), NPAD, jnp.int32)])

    zeros128 = jnp.zeros((128, 128), jnp.float32)
    ones128 = zeros128.at[:, 0].set(1.0)
    dp = _degree(ones128, dst, zeros128)
    z1, dinv = _tc_encode(
        x, dp,
        gru_W_ih.reshape(1, 192), gru_W_hh.T,
        gru_b_ih.reshape(1, 192), gru_b_hh.reshape(1, 192),
        static_W, static_b.reshape(1, 32))

    acc1 = _prop(z1, src, dst, zeros128)
    z2 = _tc_mid(acc1, z1, dinv, gcn1_W, gcn1_b.reshape(1, 128), gcn2_W)

    acc2 = _prop(z2, src, dst, zeros128)
    out = _tc_head(acc2, z2, dinv, gcn2_b.reshape(1, 64), head_W,
                   head_b.reshape(1, 1))
    return out[:, 0]


# speedup vs baseline: 2.0476x; 1.0002x over previous
import functools

import jax
import jax.numpy as jnp
from jax import lax
from jax.experimental import pallas as pl
from jax.experimental.pallas import tpu as pltpu
from jax.experimental.pallas import tpu_sc as plsc

N = 100000
E = 1600000
EP = 1638400
C = 13568
NCH = 8
CH0 = 4
NPAD = NCH * C
SHIFT = 14
DMASK = (1 << SHIFT) - 1
SP = (C + 128) // 16
WS = C // 16
ET = EP // 16
B = 2048
NBLK = ET // B
NV = B // 16

_MESH = plsc.VectorSubcoreMesh(core_axis_name="c", subcore_axis_name="s")
_DN = lax.GatherDimensionNumbers(
    offset_dims=(), collapsed_slice_dims=(0,), start_index_map=(0,))


def _lane_gather(x, idx):
    return lax.gather(x, idx[:, None], _DN, slice_sizes=(1,),
                      mode=lax.GatherScatterMode.PROMISE_IN_BOUNDS)


def _prefix_incl(x, io16):
    for k in (1, 2, 4, 8):
        sh = _lane_gather(x, jnp.maximum(io16 - k, 0))
        x = x + jnp.where(io16 >= k, sh, 0)
    return x


def _compact(vals, mi, io16):
    pref = _prefix_incl(mi, io16)
    total = pref[15]
    pos = jnp.zeros((16,), jnp.int32)
    r = io16 + 1
    for b in (8, 4, 2, 1):
        npos = pos + b
        pv = _lane_gather(pref, npos - 1)
        pos = jnp.where(pv < r, npos, pos)
    return _lane_gather(vals, jnp.minimum(pos, 15)), total


def _zero_stripe(zeros_hbm, acc, base, rows):
    for j in range(rows // 128):
        pltpu.sync_copy(zeros_hbm, acc.at[pl.ds(base + j * 128, 128)])
    rem = rows % 128
    if rem:
        pltpu.sync_copy(zeros_hbm.at[pl.ds(0, rem)],
                        acc.at[pl.ds(base + (rows // 128) * 128, rem)])


def _writeback(acc, out_hbm, s, lo):
    wbase = s * WS
    for j in range(WS // 128):
        pltpu.sync_copy(acc.at[pl.ds(wbase + j * 128, 128)],
                        out_hbm.at[pl.ds(lo + wbase + j * 128, 128)])
    wrem = WS % 128
    if wrem:
        woff = wbase + (WS // 128) * 128
        pltpu.sync_copy(acc.at[pl.ds(woff, wrem)],
                        out_hbm.at[pl.ds(lo + woff, wrem)])


@functools.partial(
    pl.kernel,
    mesh=_MESH,
    out_type=jax.ShapeDtypeStruct((NPAD, 128), jnp.float32),
    scratch_types=[
        pltpu.VMEM((2 * B,), jnp.int32),
        pltpu.VMEM((128,), jnp.int32),
        pltpu.VMEM((128,), jnp.int32),
        pltpu.VMEM((128,), jnp.int32),
        pltpu.VMEM((128, 128), jnp.float32),
        pltpu.VMEM_SHARED((C + 128, 128), jnp.float32),
        pltpu.SemaphoreType.DMA,
    ],
)
def _prop(z_hbm, epk_hbm, zeros_hbm, out_hbm,
          eblk, cbuf, csrc, cdst, rowbuf, acc, sem):
    c = lax.axis_index("c")
    s = lax.axis_index("s")
    nmy = jnp.where(c == 0, CH0, NCH - CH0)
    io16 = lax.iota(jnp.int32, 16)

    def flush(n):
        for j in range(8):
            v = cbuf[pl.ds(j * 16, 16)]
            valid = (io16 + (j * 16)) < n
            csrc[pl.ds(j * 16, 16)] = jnp.where(
                valid, (v >> SHIFT) & 0x1FFFF, 0)
            cdst[pl.ds(j * 16, 16)] = jnp.where(valid, v & DMASK, C)
        pltpu.async_copy(z_hbm.at[csrc], rowbuf, sem).wait()
        pltpu.sync_copy(rowbuf, acc.at[cdst], add=True)

    def chunk_body(kk, carry):
        @pl.when(kk < nmy)
        def _():
            chunk = 2 * kk + c
            lo = chunk * C
            _zero_stripe(zeros_hbm, acc, s * SP, SP)
            plsc.subcore_barrier()

            def blk_body(b, cur):
                poff = (s * NBLK + b) * (2 * B)
                pltpu.sync_copy(epk_hbm.at[pl.ds(poff, 2 * B)], eblk)

                def v_body(v, cur2):
                    sv = eblk[pl.ds(v * 16, 16)]
                    dv = eblk[pl.ds(B + v * 16, 16)]
                    dl = dv - lo
                    m = (dl >= 0) & (dl < C)
                    packed = (sv << SHIFT) | (dl & DMASK)
                    comp, total = _compact(packed, jnp.where(m, 1, 0), io16)
                    cbuf[pl.ds(cur2, 16)] = comp
                    cur3 = cur2 + total
                    fl = cur3 > 112

                    @pl.when(fl)
                    def _():
                        flush(cur3)

                    return jnp.where(fl, 0, cur3)

                return lax.fori_loop(0, NV, v_body, cur)

            cur = lax.fori_loop(0, NBLK, blk_body, 0)

            @pl.when(cur > 0)
            def _():
                flush(cur)

            plsc.subcore_barrier()
            _writeback(acc, out_hbm, s, lo)
            plsc.subcore_barrier()
        return carry

    lax.fori_loop(0, CH0, chunk_body, 0)


@functools.partial(
    pl.kernel,
    mesh=_MESH,
    out_type=jax.ShapeDtypeStruct((NPAD, 128), jnp.float32),
    scratch_types=[
        pltpu.VMEM((B,), jnp.int32),
        pltpu.VMEM((128,), jnp.int32),
        pltpu.VMEM((128,), jnp.int32),
        pltpu.VMEM((128, 128), jnp.float32),
        pltpu.VMEM_SHARED((C + 128, 128), jnp.float32),
    ],
)
def _degree(ones_hbm, epk_hbm, zeros_hbm, out_hbm,
            dstblk, cbuf, cdst, onesbuf, acc):
    c = lax.axis_index("c")
    s = lax.axis_index("s")
    nmy = jnp.where(c == 0, CH0, NCH - CH0)
    io16 = lax.iota(jnp.int32, 16)

    pltpu.sync_copy(ones_hbm, onesbuf)

    def flush(n):
        for j in range(8):
            v = cbuf[pl.ds(j * 16, 16)]
            valid = (io16 + (j * 16)) < n
            cdst[pl.ds(j * 16, 16)] = jnp.where(valid, v, C)
        pltpu.sync_copy(onesbuf, acc.at[cdst], add=True)

    def chunk_body(kk, carry):
        @pl.when(kk < nmy)
        def _():
            chunk = 2 * kk + c
            lo = chunk * C
            _zero_stripe(zeros_hbm, acc, s * SP, SP)
            plsc.subcore_barrier()

            def blk_body(b, cur):
                poff = (s * NBLK + b) * (2 * B) + B
                pltpu.sync_copy(epk_hbm.at[pl.ds(poff, B)], dstblk)

                def v_body(v, cur2):
                    dv = dstblk[pl.ds(v * 16, 16)]
                    dl = dv - lo
                    m = (dl >= 0) & (dl < C)
                    comp, total = _compact(dl, jnp.where(m, 1, 0), io16)
                    cbuf[pl.ds(cur2, 16)] = comp
                    cur3 = cur2 + total
                    fl = cur3 > 112

                    @pl.when(fl)
                    def _():
                        flush(cur3)

                    return jnp.where(fl, 0, cur3)

                return lax.fori_loop(0, NV, v_body, cur)

            cur = lax.fori_loop(0, NBLK, blk_body, 0)

            @pl.when(cur > 0)
            def _():
                flush(cur)

            plsc.subcore_barrier()
            _writeback(acc, out_hbm, s, lo)
            plsc.subcore_barrier()
        return carry

    lax.fori_loop(0, CH0, chunk_body, 0)


_R = 2048
_G = (N + _R - 1) // _R


def _tc_encode(x, dp, wih_t, whh_t, b_ih, b_hh, sW, sb):
    def body(x_ref, dp_ref, wih_ref, whh_ref, bih_ref, bhh_ref,
             sW_ref, sb_ref, z1_ref, dinv_ref):
        xb = x_ref[...]
        deg = dp_ref[...][:, 0] + 1.0
        dinv = lax.rsqrt(deg)
        pre = xb[:, :16] @ sW_ref[...] + sb_ref[...]
        s_enc = 0.5 * pre * (1.0 + lax.erf(pre * 0.7071067811865476))
        h = jnp.zeros((_R, 64), jnp.float32)
        for t in range(6):
            xt = xb[:, 16 + t:17 + t]
            gi = xt * wih_ref[...] + bih_ref[...]
            gh = h @ whh_ref[...] + bhh_ref[...]
            r = jax.nn.sigmoid(gi[:, :64] + gh[:, :64])
            z = jax.nn.sigmoid(gi[:, 64:128] + gh[:, 64:128])
            nn_ = jnp.tanh(gi[:, 128:] + r * gh[:, 128:])
            h = (1.0 - z) * nn_ + z * h
        hcat = jnp.concatenate(
            [h, s_enc, jnp.zeros((_R, 32), jnp.float32)], axis=1)
        z1_ref[...] = dinv[:, None] * hcat
        dinv_ref[...] = dinv[:, None]

    return pl.pallas_call(
        body,
        grid=(_G,),
        out_shape=(jax.ShapeDtypeStruct((N, 128), jnp.float32),
                   jax.ShapeDtypeStruct((N, 1), jnp.float32)),
        in_specs=[
            pl.BlockSpec((_R, 34), lambda i: (i, 0)),
            pl.BlockSpec((_R, 128), lambda i: (i, 0)),
            pl.BlockSpec((1, 192), lambda i: (0, 0)),
            pl.BlockSpec((64, 192), lambda i: (0, 0)),
            pl.BlockSpec((1, 192), lambda i: (0, 0)),
            pl.BlockSpec((1, 192), lambda i: (0, 0)),
            pl.BlockSpec((16, 32), lambda i: (0, 0)),
            pl.BlockSpec((1, 32), lambda i: (0, 0)),
        ],
        out_specs=(pl.BlockSpec((_R, 128), lambda i: (i, 0)),
                   pl.BlockSpec((_R, 1), lambda i: (i, 0))),
    )(x, dp, wih_t, whh_t, b_ih, b_hh, sW, sb)


def _tc_mid(acc1, z1, dinv, W1, b1, W2):
    def body(acc_ref, z1_ref, dinv_ref, W1_ref, b1_ref, W2_ref, z2_ref):
        p1 = dinv_ref[...] * (acc_ref[...][:, :96] + z1_ref[...][:, :96])
        h1 = jnp.maximum(p1 @ W1_ref[...] + b1_ref[...], 0.0)
        z2 = dinv_ref[...] * (h1 @ W2_ref[...])
        z2_ref[...] = jnp.concatenate(
            [z2, jnp.zeros((_R, 64), jnp.float32)], axis=1)

    return pl.pallas_call(
        body,
        grid=(_G,),
        out_shape=jax.ShapeDtypeStruct((N, 128), jnp.float32),
        in_specs=[
            pl.BlockSpec((_R, 128), lambda i: (i, 0)),
            pl.BlockSpec((_R, 128), lambda i: (i, 0)),
            pl.BlockSpec((_R, 1), lambda i: (i, 0)),
            pl.BlockSpec((96, 128), lambda i: (0, 0)),
            pl.BlockSpec((1, 128), lambda i: (0, 0)),
            pl.BlockSpec((128, 64), lambda i: (0, 0)),
        ],
        out_specs=pl.BlockSpec((_R, 128), lambda i: (i, 0)),
    )(acc1, z1, dinv, W1, b1, W2)


def _tc_head(acc2, z2, dinv, b2, head_W, head_b):
    def body(acc_ref, z2_ref, dinv_ref, b2_ref, hw_ref, hb_ref, out_ref):
        h2 = jnp.maximum(
            dinv_ref[...] * (acc_ref[...][:, :64] + z2_ref[...][:, :64])
            + b2_ref[...], 0.0)
        out_ref[...] = jax.nn.sigmoid(h2 @ hw_ref[...] + hb_ref[0, 0])

    return pl.pallas_call(
        body,
        grid=(_G,),
        out_shape=jax.ShapeDtypeStruct((N, 1), jnp.float32),
        in_specs=[
            pl.BlockSpec((_R, 128), lambda i: (i, 0)),
            pl.BlockSpec((_R, 128), lambda i: (i, 0)),
            pl.BlockSpec((_R, 1), lambda i: (i, 0)),
            pl.BlockSpec((1, 64), lambda i: (0, 0)),
            pl.BlockSpec((64, 1), lambda i: (0, 0)),
            pl.BlockSpec((1, 1), lambda i: (0, 0)),
        ],
        out_specs=pl.BlockSpec((_R, 1), lambda i: (i, 0)),
    )(acc2, z2, dinv, b2, head_W, head_b)


def kernel(x, edge_index, gru_W_ih, gru_W_hh, gru_b_ih, gru_b_hh, static_W,
           static_b, gcn1_W, gcn1_b, gcn2_W, gcn2_b, head_W, head_b):
    pad = EP - E
    src = jnp.concatenate([edge_index[0], jnp.zeros((pad,), jnp.int32)])
    dst = jnp.concatenate([edge_index[1], jnp.full((pad,), NPAD, jnp.int32)])
    epk = jnp.stack([src.reshape(16, NBLK, B), dst.reshape(16, NBLK, B)],
                    axis=2).reshape(-1)

    zeros128 = jnp.zeros((128, 128), jnp.float32)
    ones128 = zeros128.at[:, 0].set(1.0)
    dp = _degree(ones128, epk, zeros128)
    z1, dinv = _tc_encode(
        x, dp,
        gru_W_ih.reshape(1, 192), gru_W_hh.T,
        gru_b_ih.reshape(1, 192), gru_b_hh.reshape(1, 192),
        static_W, static_b.reshape(1, 32))

    acc1 = _prop(z1, epk, zeros128)
    z2 = _tc_mid(acc1, z1, dinv, gcn1_W, gcn1_b.reshape(1, 128), gcn2_W)

    acc2 = _prop(z2, epk, zeros128)
    out = _tc_head(acc2, z2, dinv, gcn2_b.reshape(1, 64), head_W,
                   head_b.reshape(1, 1))
    return out[:, 0]
